# baseline (speedup 1.0000x reference)
import functools
import numpy as np
import jax
import jax.numpy as jnp
from jax import lax
from jax.experimental import pallas as pl
from jax.experimental.pallas import tpu as pltpu
from jax.experimental.pallas import tpu_sc as plsc

_IDX = [162, 1098, 1377]

ROWS = 16384
COLS = 2048
TC_ROWS = 9216
SC_ROWS = ROWS - TC_ROWS
BLOCK_ROWS = 1024

NC, NS = 2, 16
NW = NC * NS
ROWS_PER_W = SC_ROWS // NW
CHUNK = 8
NCHUNKS = ROWS_PER_W // CHUNK
NBUF = 4


def _mask_copy_kernel(img_ref, out_ref):
    x = img_ref[...]
    cols = lax.broadcasted_iota(jnp.int32, x.shape, dimension=1)
    keep = jnp.ones(x.shape, jnp.bool_)
    for c in _IDX:
        keep = keep & (cols != c)
    out_ref[...] = jnp.where(keep, x, jnp.float32(0.0))


def _sc_body(img_hbm, out_hbm, buf, sems):
    wid = lax.axis_index("s") * NC + lax.axis_index("c")
    base = TC_ROWS + wid * ROWS_PER_W
    obase = wid * ROWS_PER_W
    lane = lax.iota(jnp.int32, 16)

    def load(g, slot):
        return pltpu.make_async_copy(
            img_hbm.at[pl.ds(base + g * CHUNK, CHUNK), :],
            buf.at[slot],
            sems.at[slot],
        )

    def store(g, slot):
        return pltpu.make_async_copy(
            buf.at[slot],
            out_hbm.at[pl.ds(obase + g * CHUNK, CHUNK), :],
            sems.at[NBUF + slot],
        )

    for b in range(NBUF):
        load(b, b).start()

    def outer(i, carry):
        for b in range(NBUF):
            g = i * NBUF + b
            load(g, b).wait()
            for c in _IDX:
                c0 = (c // 16) * 16
                off = c % 16
                for r in range(CHUNK):
                    v = buf[b, r, pl.ds(c0, 16)]
                    buf[b, r, pl.ds(c0, 16)] = jnp.where(
                        lane == off, jnp.float32(0.0), v
                    )
            store(g, b).start()

            @pl.when(g + NBUF < NCHUNKS)
            def _():
                store(g, b).wait()
                load(g + NBUF, b).start()

        return carry

    lax.fori_loop(0, NCHUNKS // NBUF, outer, 0)

    for b in range(NBUF):
        store(NCHUNKS - NBUF + b, b).wait()


def kernel(img):
    tc_out = pl.pallas_call(
        _mask_copy_kernel,
        grid=(TC_ROWS // BLOCK_ROWS,),
        in_specs=[pl.BlockSpec((BLOCK_ROWS, COLS), lambda i: (i, 0))],
        out_specs=pl.BlockSpec((BLOCK_ROWS, COLS), lambda i: (i, 0)),
        out_shape=jax.ShapeDtypeStruct((TC_ROWS, COLS), img.dtype),
    )(img)

    mesh = plsc.VectorSubcoreMesh(core_axis_name="c", subcore_axis_name="s")
    sc_out = functools.partial(
        pl.kernel,
        mesh=mesh,
        out_type=jax.ShapeDtypeStruct((SC_ROWS, COLS), jnp.float32),
        scratch_types=[
            pltpu.VMEM((NBUF, CHUNK, COLS), jnp.float32),
            pltpu.SemaphoreType.DMA((2 * NBUF,)),
        ],
    )(_sc_body)(img)

    return jnp.concatenate([tc_out, sc_out], axis=0)


# speedup vs baseline: 1.6725x; 1.6725x over previous
import functools
import numpy as np
import jax
import jax.numpy as jnp
from jax import lax
from jax.experimental import pallas as pl
from jax.experimental.pallas import tpu as pltpu
from jax.experimental.pallas import tpu_sc as plsc

_IDX = [162, 1098, 1377]

ROWS = 16384
COLS = 2048
NC, NS = 2, 16
NW = NC * NS
ROWS_PER_W = ROWS // NW
CHUNK = 16
NCHUNKS = ROWS_PER_W // CHUNK
NBUF = 2


def _sc_body(img_hbm, out_hbm, buf, sems):
    wid = lax.axis_index("s") * NC + lax.axis_index("c")
    base = wid * ROWS_PER_W
    lane = lax.iota(jnp.int32, 16)

    def load(g, slot):
        return pltpu.make_async_copy(
            img_hbm.at[pl.ds(base + g * CHUNK, CHUNK), :],
            buf.at[slot],
            sems.at[slot],
        )

    def store(g, slot):
        return pltpu.make_async_copy(
            buf.at[slot],
            out_hbm.at[pl.ds(base + g * CHUNK, CHUNK), :],
            sems.at[NBUF + slot],
        )

    for b in range(NBUF):
        load(b, b).start()

    def outer(i, carry):
        for b in range(NBUF):
            g = i * NBUF + b
            load(g, b).wait()
            for c in _IDX:
                c0 = (c // 16) * 16
                off = c % 16
                for r in range(CHUNK):
                    v = buf[b, r, pl.ds(c0, 16)]
                    buf[b, r, pl.ds(c0, 16)] = jnp.where(
                        lane == off, jnp.float32(0.0), v
                    )
            store(g, b).start()

            @pl.when(g + NBUF < NCHUNKS)
            def _():
                store(g, b).wait()
                load(g + NBUF, b).start()

        return carry

    lax.fori_loop(0, NCHUNKS // NBUF, outer, 0)

    for b in range(NBUF):
        store(NCHUNKS - NBUF + b, b).wait()


def kernel(img):
    mesh = plsc.VectorSubcoreMesh(core_axis_name="c", subcore_axis_name="s")
    k = functools.partial(
        pl.kernel,
        mesh=mesh,
        out_type=jax.ShapeDtypeStruct((ROWS, COLS), jnp.float32),
        scratch_types=[
            pltpu.VMEM((NBUF, CHUNK, COLS), jnp.float32),
            pltpu.SemaphoreType.DMA((2 * NBUF,)),
        ],
    )(_sc_body)
    return k(img)


# speedup vs baseline: 2.2506x; 1.3456x over previous
import numpy as np
import jax
import jax.numpy as jnp
from jax.experimental import pallas as pl
from jax.experimental.pallas import tpu as pltpu


def _disabled_tofs(tof_count, min_c, max_c, neighbor_p, seed=0):
    rng = np.random.default_rng(seed)
    count = int(rng.integers(min_c, max_c + 1))
    tof_list = rng.permutation(tof_count)
    first = int(rng.integers(1, tof_count))
    disabled = [first]
    tof_list = tof_list[tof_list != first]
    for _ in range(count - 1):
        r = float(rng.random())
        if r < neighbor_p:
            if r < neighbor_p / 2.0:
                offsets = (1, -1)
            else:
                offsets = (tof_count // 2, -(tof_count // 2))
            appended = False
            for d in list(disabled):
                for off in offsets:
                    cand = d + off
                    if cand in tof_list:
                        tof_list = tof_list[tof_list != cand]
                        disabled.append(int(cand))
                        appended = True
                        break
                if appended:
                    break
            if not appended:
                new = int(tof_list[0])
                tof_list = tof_list[tof_list != new]
                disabled.append(new)
        else:
            new = int(tof_list[0])
            tof_list = tof_list[tof_list != new]
            disabled.append(new)
    return sorted(int(x) for x in disabled)


_IDX = _disabled_tofs(2048, 1, 3, 0.5)

BLOCK_ROWS = 1024


def _mask_copy_kernel(img_ref, out_ref):
    x = img_ref[...]
    cols = jax.lax.broadcasted_iota(jnp.int32, x.shape, dimension=1)
    keep = jnp.ones(x.shape, jnp.bool_)
    for c in _IDX:
        keep = keep & (cols != c)
    out_ref[...] = jnp.where(keep, x, jnp.float32(0.0))


def kernel(img):
    n_rows, n_cols = img.shape
    grid = (n_rows // BLOCK_ROWS,)
    return pl.pallas_call(
        _mask_copy_kernel,
        grid=grid,
        in_specs=[pl.BlockSpec((BLOCK_ROWS, n_cols), lambda i: (i, 0))],
        out_specs=pl.BlockSpec((BLOCK_ROWS, n_cols), lambda i: (i, 0)),
        out_shape=jax.ShapeDtypeStruct((n_rows, n_cols), img.dtype),
        compiler_params=pltpu.CompilerParams(
            vmem_limit_bytes=128 * 1024 * 1024,
        ),
    )(img)
